# baseline (speedup 1.0000x reference)
import jax
import jax.numpy as jnp
from jax.experimental import pallas as pl

_G = 64


def _gcn_block(x_ref, adj_ref, w_ref, b_ref, out_ref):
    g, n, d = x_ref.shape
    o = w_ref.shape[1]
    x = x_ref[...].reshape(g * n, d)
    xw = jnp.dot(x, w_ref[...], preferred_element_type=jnp.float32)
    a_hat = adj_ref[...].astype(jnp.float32) + jnp.eye(n, dtype=jnp.float32)[None]
    dinv = jax.lax.rsqrt(jnp.sum(a_hat, axis=-1))
    xwn = xw.reshape(g, n, o) * dinv[:, :, None]
    agg = jax.lax.dot_general(
        a_hat, xwn, (((2,), (1,)), ((0,), (0,))),
        preferred_element_type=jnp.float32)
    out_ref[...] = agg * dinv[:, :, None] + b_ref[0][None, None, :]


def kernel(node_states, adj, W_gnn, b_gnn):
    b, n, d = node_states.shape
    o = W_gnn.shape[1]
    out = pl.pallas_call(
        _gcn_block,
        grid=(b // _G,),
        in_specs=[
            pl.BlockSpec((_G, n, d), lambda i: (i, 0, 0)),
            pl.BlockSpec((_G, n, n), lambda i: (i, 0, 0)),
            pl.BlockSpec((d, o), lambda i: (0, 0)),
            pl.BlockSpec((1, o), lambda i: (0, 0)),
        ],
        out_specs=pl.BlockSpec((_G, n, o), lambda i: (i, 0, 0)),
        out_shape=jax.ShapeDtypeStruct((b, n, o), jnp.float32),
    )(node_states, adj, W_gnn, b_gnn.reshape(1, o))
    return out.reshape(b, n * o)


# speedup vs baseline: 1.4968x; 1.4968x over previous
import jax
import jax.numpy as jnp
from jax.experimental import pallas as pl

_G = 64


def _gcn_block(x_ref, adj_ref, w_ref, b_ref, out_ref):
    g, n, d = x_ref.shape
    o = w_ref.shape[1]
    x = x_ref[...].reshape(g * n, d)
    xw = jnp.dot(x, w_ref[...], preferred_element_type=jnp.float32)
    adj_f = adj_ref[...].astype(jnp.float32)
    dinv = jax.lax.rsqrt(jnp.sum(adj_f, axis=-1) + 1.0)
    xwn = xw.reshape(g, n, o) * dinv[:, :, None]
    agg = jax.lax.dot_general(
        adj_f, xwn, (((2,), (1,)), ((0,), (0,))),
        preferred_element_type=jnp.float32) + xwn
    out = agg * dinv[:, :, None] + b_ref[0][None, None, :]
    out_ref[...] = out.reshape(g, n * o)


def kernel(node_states, adj, W_gnn, b_gnn):
    b, n, d = node_states.shape
    o = W_gnn.shape[1]
    out = pl.pallas_call(
        _gcn_block,
        grid=(b // _G,),
        in_specs=[
            pl.BlockSpec((_G, n, d), lambda i: (i, 0, 0)),
            pl.BlockSpec((_G, n, n), lambda i: (i, 0, 0)),
            pl.BlockSpec((d, o), lambda i: (0, 0)),
            pl.BlockSpec((1, o), lambda i: (0, 0)),
        ],
        out_specs=pl.BlockSpec((_G, n * o), lambda i: (i, 0)),
        out_shape=jax.ShapeDtypeStruct((b, n * o), jnp.float32),
    )(node_states, adj, W_gnn, b_gnn.reshape(1, o))
    return out


# speedup vs baseline: 1.6895x; 1.1288x over previous
import jax
import jax.numpy as jnp
from jax.experimental import pallas as pl

_G = 128


def _gcn_block(x_ref, adj_ref, w_ref, b_ref, out_ref):
    g, n, d = x_ref.shape
    o = w_ref.shape[1]
    x = x_ref[...].reshape(g * n, d)
    xw = jnp.dot(x, w_ref[...], preferred_element_type=jnp.float32)
    adj_f = adj_ref[...].astype(jnp.float32)
    dinv = jax.lax.rsqrt(jnp.sum(adj_f, axis=-1) + 1.0)
    xwn = xw.reshape(g, n, o) * dinv[:, :, None]
    agg = jax.lax.dot_general(
        adj_f, xwn, (((2,), (1,)), ((0,), (0,))),
        preferred_element_type=jnp.float32) + xwn
    out = agg * dinv[:, :, None] + b_ref[0][None, None, :]
    out_ref[...] = out.reshape(g, n * o)


def kernel(node_states, adj, W_gnn, b_gnn):
    b, n, d = node_states.shape
    o = W_gnn.shape[1]
    out = pl.pallas_call(
        _gcn_block,
        grid=(b // _G,),
        in_specs=[
            pl.BlockSpec((_G, n, d), lambda i: (i, 0, 0)),
            pl.BlockSpec((_G, n, n), lambda i: (i, 0, 0)),
            pl.BlockSpec((d, o), lambda i: (0, 0)),
            pl.BlockSpec((1, o), lambda i: (0, 0)),
        ],
        out_specs=pl.BlockSpec((_G, n * o), lambda i: (i, 0)),
        out_shape=jax.ShapeDtypeStruct((b, n * o), jnp.float32),
    )(node_states, adj, W_gnn, b_gnn.reshape(1, o))
    return out


# speedup vs baseline: 1.7848x; 1.0564x over previous
import jax
import jax.numpy as jnp
from jax.experimental import pallas as pl

_G = 256


def _gcn_block(x_ref, adj_ref, w_ref, b_ref, out_ref):
    g, n, d = x_ref.shape
    o = w_ref.shape[1]
    x = x_ref[...].reshape(g * n, d)
    xw = jnp.dot(x, w_ref[...], preferred_element_type=jnp.float32)
    adj_f = adj_ref[...].astype(jnp.float32)
    dinv = jax.lax.rsqrt(jnp.sum(adj_f, axis=-1) + 1.0)
    xwn = xw.reshape(g, n, o) * dinv[:, :, None]
    agg = jax.lax.dot_general(
        adj_f, xwn, (((2,), (1,)), ((0,), (0,))),
        preferred_element_type=jnp.float32) + xwn
    out = agg * dinv[:, :, None] + b_ref[0][None, None, :]
    out_ref[...] = out.reshape(g, n * o)


def kernel(node_states, adj, W_gnn, b_gnn):
    b, n, d = node_states.shape
    o = W_gnn.shape[1]
    out = pl.pallas_call(
        _gcn_block,
        grid=(b // _G,),
        in_specs=[
            pl.BlockSpec((_G, n, d), lambda i: (i, 0, 0)),
            pl.BlockSpec((_G, n, n), lambda i: (i, 0, 0)),
            pl.BlockSpec((d, o), lambda i: (0, 0)),
            pl.BlockSpec((1, o), lambda i: (0, 0)),
        ],
        out_specs=pl.BlockSpec((_G, n * o), lambda i: (i, 0)),
        out_shape=jax.ShapeDtypeStruct((b, n * o), jnp.float32),
    )(node_states, adj, W_gnn, b_gnn.reshape(1, o))
    return out
